# baseline (speedup 1.0000x reference)
import functools

import jax
import jax.numpy as jnp
from jax import lax
from jax.experimental import pallas as pl
from jax.experimental.pallas import tpu as pltpu

DT = 0.5
KNN = 30


def _rb(x):
    return x.astype(jnp.bfloat16).astype(jnp.float32)


def _mlp_body(u_ref, s_ref, w1_ref, b1_ref, w2t_ref, b2_ref, abg_ref,
              uv_ref, sv_ref):
    u = u_ref[...]
    s = s_ref[...]
    h = _rb(u) * _rb(w1_ref[0:1, :]) + _rb(s) * _rb(w1_ref[1:2, :]) \
        + b1_ref[...]
    h = jnp.where(h >= 0.0, h, 0.01 * h)
    hb = _rb(h)
    outs = []
    for m in range(3):
        z = jnp.sum(hb * _rb(w2t_ref[m:m + 1, :]), axis=1, keepdims=True) \
            + b2_ref[0, m]
        outs.append(1.0 / (1.0 + jnp.exp(-z)))
    alphas = outs[0] * abg_ref[0, 0]
    beta = outs[1] * abg_ref[0, 1]
    gamma = outs[2] * abg_ref[0, 2]
    u1 = u + (alphas - beta * u) * DT
    s1 = s + (beta * u - gamma * s) * DT
    uv_ref[...] = u1 - u
    sv_ref[...] = s1 - s


def _cost_body(p1_ref, p2_ref, u0r_ref, s0r_ref,
               q1_ref, q2_ref, qu_ref, qs_ref, quv_ref, qsv_ref,
               out_ref, d2_ref, *, q_blk, n):
    i = pl.program_id(0)
    lanes = 128
    nchunks = n // lanes
    q1 = q1_ref[...]
    q2 = q2_ref[...]
    p1 = p1_ref[...]
    p2 = p2_ref[...]
    q1b, q2b = _rb(q1), _rb(q2)
    p1b, p2b = _rb(p1), _rb(p2)
    qsq = q1 * q1 + q2 * q2
    psq = p1 * p1 + p2 * p2
    inf = jnp.float32(jnp.inf)

    m1 = jnp.full((q_blk, lanes), inf, jnp.float32)
    m2, m3, m4 = m1, m1, m1
    jarg = jnp.zeros((q_blk, lanes), jnp.int32)
    lane_iota = lax.broadcasted_iota(jnp.int32, (q_blk, lanes), 1)
    for c in range(nchunks):
        sl = slice(c * lanes, (c + 1) * lanes)
        d2c = (qsq + psq[:, sl]) - 2.0 * (q1b * p1b[:, sl] + q2b * p2b[:, sl])
        d2_ref[:, sl] = d2c
        jarg = jnp.where(d2c < m1, lane_iota + (c * lanes), jarg)
        t1 = jnp.maximum(m1, d2c)
        m1 = jnp.minimum(m1, d2c)
        t2 = jnp.maximum(m2, t1)
        m2 = jnp.minimum(m2, t1)
        t3 = jnp.maximum(m3, t2)
        m3 = jnp.minimum(m3, t2)
        m4 = jnp.minimum(m4, t3)

    m0 = jnp.min(m1, axis=1, keepdims=True)
    jmin = jnp.min(jnp.where(m1 == m0, jarg, jnp.int32(n)), axis=1,
                   keepdims=True)

    def dstep(_, t):
        w = jnp.where(m1 > t, m1, inf)
        w = jnp.minimum(w, jnp.where(m2 > t, m2, inf))
        w = jnp.minimum(w, jnp.where(m3 > t, m3, inf))
        w = jnp.minimum(w, jnp.where(m4 > t, m4, inf))
        return jnp.min(w, axis=1, keepdims=True)

    tau = lax.fori_loop(0, KNN, dstep, m0)

    jj = lax.broadcasted_iota(jnp.int32, (q_blk, n), 1)
    d = d2_ref[...]
    cnt = jnp.sum(jnp.where(d <= tau, 1.0, 0.0), axis=1, keepdims=True)

    def rcond(state):
        _, act = state
        return jnp.max(act) > 0.5

    def rbody(state):
        t, act = state
        dd = d2_ref[...]
        tmax = jnp.max(jnp.where(dd < t, dd, -inf), axis=1, keepdims=True)
        cnew = jnp.sum(jnp.where(dd <= tmax, 1.0, 0.0), axis=1, keepdims=True)
        ok = (act > 0.5) & (cnew >= 31.0)
        t = jnp.where(ok, tmax, t)
        act = jnp.where((act > 0.5) & (cnew > 31.0), 1.0, 0.0)
        return t, act

    act0 = jnp.where(cnt > 31.0, 1.0, 0.0)
    tau, _ = lax.while_loop(rcond, rbody, (tau, act0))

    sel = (d <= tau) & (jj != jmin)

    quv = quv_ref[...]
    qsv = qsv_ref[...]
    vnorm2 = quv * quv + qsv * qsv
    unv = u0r_ref[...] - qu_ref[...]
    snv = s0r_ref[...] - qs_ref[...]
    den2 = (unv * unv + snv * snv) * vnorm2
    num = unv * quv + snv * qsv
    r = jnp.where(den2 != 0.0, (num * jnp.abs(num)) / den2, jnp.float32(4.0))
    r = jnp.where(sel, r, jnp.float32(-9.0))
    rmax = jnp.max(r, axis=1)
    cmax = jnp.where(rmax > 1.5, jnp.float32(1.0),
                     jnp.sign(rmax) * jnp.sqrt(jnp.abs(rmax)))
    blk = jnp.sum(1.0 - cmax)

    @pl.when(i == 0)
    def _():
        out_ref[0, 0] = 0.0

    out_ref[0, 0] += blk


def kernel(u0, s0, alpha0, beta0, gamma0, embedding1, embedding2,
           W1, b1, W2, b2):
    n = u0.shape[0]
    hid = W1.shape[1]

    b_mlp = 2048
    abg = jnp.stack([alpha0[0], beta0[0], gamma0[0]])[None, :]
    uv, sv = pl.pallas_call(
        _mlp_body,
        grid=(n // b_mlp,),
        in_specs=[
            pl.BlockSpec((b_mlp, 1), lambda i: (i, 0)),
            pl.BlockSpec((b_mlp, 1), lambda i: (i, 0)),
            pl.BlockSpec((2, hid), lambda i: (0, 0)),
            pl.BlockSpec((1, hid), lambda i: (0, 0)),
            pl.BlockSpec((3, hid), lambda i: (0, 0)),
            pl.BlockSpec((1, 3), lambda i: (0, 0)),
            pl.BlockSpec((1, 3), lambda i: (0, 0)),
        ],
        out_specs=[
            pl.BlockSpec((b_mlp, 1), lambda i: (i, 0)),
            pl.BlockSpec((b_mlp, 1), lambda i: (i, 0)),
        ],
        out_shape=[
            jax.ShapeDtypeStruct((n, 1), jnp.float32),
            jax.ShapeDtypeStruct((n, 1), jnp.float32),
        ],
    )(u0[:, None], s0[:, None], W1, b1[None, :], W2.T, b2[None, :], abg)

    q_blk = 256
    body = functools.partial(_cost_body, q_blk=q_blk, n=n)
    row = lambda a: a[None, :]
    col = lambda a: a[:, None]
    total = pl.pallas_call(
        body,
        grid=(n // q_blk,),
        in_specs=[
            pl.BlockSpec((1, n), lambda i: (0, 0)),
            pl.BlockSpec((1, n), lambda i: (0, 0)),
            pl.BlockSpec((1, n), lambda i: (0, 0)),
            pl.BlockSpec((1, n), lambda i: (0, 0)),
            pl.BlockSpec((q_blk, 1), lambda i: (i, 0)),
            pl.BlockSpec((q_blk, 1), lambda i: (i, 0)),
            pl.BlockSpec((q_blk, 1), lambda i: (i, 0)),
            pl.BlockSpec((q_blk, 1), lambda i: (i, 0)),
            pl.BlockSpec((q_blk, 1), lambda i: (i, 0)),
            pl.BlockSpec((q_blk, 1), lambda i: (i, 0)),
        ],
        out_specs=pl.BlockSpec(memory_space=pltpu.SMEM),
        out_shape=jax.ShapeDtypeStruct((1, 1), jnp.float32),
        scratch_shapes=[pltpu.VMEM((q_blk, n), jnp.float32)],
    )(row(embedding1), row(embedding2), row(u0), row(s0),
      col(embedding1), col(embedding2), col(u0), col(s0), uv, sv)

    return total[0, 0] / jnp.float32(n)


# speedup vs baseline: 1.2693x; 1.2693x over previous
import functools

import jax
import jax.numpy as jnp
from jax import lax
from jax.experimental import pallas as pl
from jax.experimental.pallas import tpu as pltpu

DT = 0.5
KNN = 30


def _rb(x):
    return x.astype(jnp.bfloat16).astype(jnp.float32)


def _mlp_body(u_ref, s_ref, w1_ref, b1_ref, w2t_ref, b2_ref, abg_ref,
              uv_ref, sv_ref):
    u = u_ref[...]
    s = s_ref[...]
    h = _rb(u) * _rb(w1_ref[0:1, :]) + _rb(s) * _rb(w1_ref[1:2, :]) \
        + b1_ref[...]
    h = jnp.where(h >= 0.0, h, 0.01 * h)
    hb = _rb(h)
    outs = []
    for m in range(3):
        z = jnp.sum(hb * _rb(w2t_ref[m:m + 1, :]), axis=1, keepdims=True) \
            + b2_ref[0, m]
        outs.append(1.0 / (1.0 + jnp.exp(-z)))
    alphas = outs[0] * abg_ref[0, 0]
    beta = outs[1] * abg_ref[0, 1]
    gamma = outs[2] * abg_ref[0, 2]
    u1 = u + (alphas - beta * u) * DT
    s1 = s + (beta * u - gamma * s) * DT
    uv_ref[...] = u1 - u
    sv_ref[...] = s1 - s


def _cost_body(p1_ref, p2_ref, u0r_ref, s0r_ref,
               q1_ref, q2_ref, qu_ref, qs_ref, quv_ref, qsv_ref,
               out_ref, d2_ref, *, q_blk, n):
    i = pl.program_id(0)
    lanes = 128
    nchunks = n // lanes
    q1 = q1_ref[...]
    q2 = q2_ref[...]
    p1 = p1_ref[...]
    p2 = p2_ref[...]
    q1b, q2b = _rb(q1), _rb(q2)
    p1b, p2b = _rb(p1), _rb(p2)
    qsq = q1 * q1 + q2 * q2
    psq = p1 * p1 + p2 * p2
    inf = jnp.float32(jnp.inf)

    m1 = jnp.full((q_blk, lanes), inf, jnp.float32)
    m2, m3, m4 = m1, m1, m1
    jarg = jnp.zeros((q_blk, lanes), jnp.int32)
    lane_iota = lax.broadcasted_iota(jnp.int32, (q_blk, lanes), 1)
    for c in range(nchunks):
        sl = slice(c * lanes, (c + 1) * lanes)
        d2c = (qsq + psq[:, sl]) - 2.0 * (q1b * p1b[:, sl] + q2b * p2b[:, sl])
        d2_ref[:, sl] = d2c
        jarg = jnp.where(d2c < m1, lane_iota + (c * lanes), jarg)
        t1 = jnp.maximum(m1, d2c)
        m1 = jnp.minimum(m1, d2c)
        t2 = jnp.maximum(m2, t1)
        m2 = jnp.minimum(m2, t1)
        t3 = jnp.maximum(m3, t2)
        m3 = jnp.minimum(m3, t2)
        m4 = jnp.minimum(m4, t3)

    m0 = jnp.min(m1, axis=1, keepdims=True)
    jmin = jnp.min(jnp.where(m1 == m0, jarg, jnp.int32(n)), axis=1,
                   keepdims=True)

    def dstep(_, t):
        w = jnp.where(m3 > t, m3, jnp.where(m4 > t, m4, inf))
        w = jnp.where(m2 > t, m2, w)
        w = jnp.where(m1 > t, m1, w)
        return jnp.min(w, axis=1, keepdims=True)

    tau = lax.fori_loop(0, KNN, dstep, m0)

    jj = lax.broadcasted_iota(jnp.int32, (q_blk, n), 1)
    d = d2_ref[...]
    cnt = jnp.sum(jnp.where(d <= tau, 1.0, 0.0), axis=1, keepdims=True)

    def rcond(state):
        _, act = state
        return jnp.max(act) > 0.5

    def rbody(state):
        t, act = state
        dd = d2_ref[...]
        tmax = jnp.max(jnp.where(dd < t, dd, -inf), axis=1, keepdims=True)
        cnew = jnp.sum(jnp.where(dd <= tmax, 1.0, 0.0), axis=1, keepdims=True)
        ok = (act > 0.5) & (cnew >= 31.0)
        t = jnp.where(ok, tmax, t)
        act = jnp.where((act > 0.5) & (cnew > 31.0), 1.0, 0.0)
        return t, act

    act0 = jnp.where(cnt > 31.0, 1.0, 0.0)
    tau, _ = lax.while_loop(rcond, rbody, (tau, act0))

    sel = (d <= tau) & (jj != jmin)

    quv = quv_ref[...]
    qsv = qsv_ref[...]
    vnorm2 = quv * quv + qsv * qsv
    unv = u0r_ref[...] - qu_ref[...]
    snv = s0r_ref[...] - qs_ref[...]
    den2 = (unv * unv + snv * snv) * vnorm2
    num = unv * quv + snv * qsv
    r = jnp.where(den2 != 0.0, (num * jnp.abs(num)) / den2, jnp.float32(4.0))
    r = jnp.where(sel, r, jnp.float32(-9.0))
    rmax = jnp.max(r, axis=1)
    cmax = jnp.where(rmax > 1.5, jnp.float32(1.0),
                     jnp.sign(rmax) * jnp.sqrt(jnp.abs(rmax)))
    blk = jnp.sum(1.0 - cmax)

    @pl.when(i == 0)
    def _():
        out_ref[0, 0] = 0.0

    out_ref[0, 0] += blk


def kernel(u0, s0, alpha0, beta0, gamma0, embedding1, embedding2,
           W1, b1, W2, b2):
    n = u0.shape[0]
    hid = W1.shape[1]

    b_mlp = 2048
    abg = jnp.stack([alpha0[0], beta0[0], gamma0[0]])[None, :]
    uv, sv = pl.pallas_call(
        _mlp_body,
        grid=(n // b_mlp,),
        in_specs=[
            pl.BlockSpec((b_mlp, 1), lambda i: (i, 0)),
            pl.BlockSpec((b_mlp, 1), lambda i: (i, 0)),
            pl.BlockSpec((2, hid), lambda i: (0, 0)),
            pl.BlockSpec((1, hid), lambda i: (0, 0)),
            pl.BlockSpec((3, hid), lambda i: (0, 0)),
            pl.BlockSpec((1, 3), lambda i: (0, 0)),
            pl.BlockSpec((1, 3), lambda i: (0, 0)),
        ],
        out_specs=[
            pl.BlockSpec((b_mlp, 1), lambda i: (i, 0)),
            pl.BlockSpec((b_mlp, 1), lambda i: (i, 0)),
        ],
        out_shape=[
            jax.ShapeDtypeStruct((n, 1), jnp.float32),
            jax.ShapeDtypeStruct((n, 1), jnp.float32),
        ],
    )(u0[:, None], s0[:, None], W1, b1[None, :], W2.T, b2[None, :], abg)

    q_blk = 128
    body = functools.partial(_cost_body, q_blk=q_blk, n=n)
    row = lambda a: a[None, :]
    col = lambda a: a[:, None]
    total = pl.pallas_call(
        body,
        grid=(n // q_blk,),
        in_specs=[
            pl.BlockSpec((1, n), lambda i: (0, 0)),
            pl.BlockSpec((1, n), lambda i: (0, 0)),
            pl.BlockSpec((1, n), lambda i: (0, 0)),
            pl.BlockSpec((1, n), lambda i: (0, 0)),
            pl.BlockSpec((q_blk, 1), lambda i: (i, 0)),
            pl.BlockSpec((q_blk, 1), lambda i: (i, 0)),
            pl.BlockSpec((q_blk, 1), lambda i: (i, 0)),
            pl.BlockSpec((q_blk, 1), lambda i: (i, 0)),
            pl.BlockSpec((q_blk, 1), lambda i: (i, 0)),
            pl.BlockSpec((q_blk, 1), lambda i: (i, 0)),
        ],
        out_specs=pl.BlockSpec(memory_space=pltpu.SMEM),
        out_shape=jax.ShapeDtypeStruct((1, 1), jnp.float32),
        scratch_shapes=[pltpu.VMEM((q_blk, n), jnp.float32)],
    )(row(embedding1), row(embedding2), row(u0), row(s0),
      col(embedding1), col(embedding2), col(u0), col(s0), uv, sv)

    return total[0, 0] / jnp.float32(n)
